# baseline (speedup 1.0000x reference)
import jax
import jax.numpy as jnp
from jax.experimental import pallas as pl
from jax.experimental.pallas import tpu as pltpu


def _agg_body(a_ref, h_ref, u_ref, s_ref):
    i = pl.program_id(0)

    @pl.when(i == 0)
    def _init_s():
        s_ref[...] = jnp.zeros_like(s_ref)

    a = a_ref[...]
    u_ref[...] = jnp.dot(
        a.astype(jnp.bfloat16), h_ref[...], preferred_element_type=jnp.float32
    )
    s_ref[...] += jnp.sum(a)[None, None]


def _proj_body(h_ref, u_ref, wa_ref, wb_ref, s_ref, o_ref):
    inv = 1.0 / s_ref[0, 0]
    o = jnp.dot(h_ref[...], wa_ref[...], preferred_element_type=jnp.float32)
    o += jnp.dot(u_ref[...], wb_ref[...], preferred_element_type=jnp.float32) * inv
    o_ref[...] = jnp.maximum(o, 0.0)


def kernel(h, A, W):
    n, d = h.shape
    bi = 400

    h16 = h.astype(jnp.bfloat16)
    u, s = pl.pallas_call(
        _agg_body,
        grid=(n // bi,),
        in_specs=[
            pl.BlockSpec((bi, n), lambda i: (i, 0)),
            pl.BlockSpec((n, d), lambda i: (0, 0)),
        ],
        out_specs=[
            pl.BlockSpec((bi, d), lambda i: (i, 0)),
            pl.BlockSpec((1, 1), lambda i: (0, 0)),
        ],
        out_shape=[
            jax.ShapeDtypeStruct((n, d), jnp.float32),
            jax.ShapeDtypeStruct((1, 1), jnp.float32),
        ],
        compiler_params=pltpu.CompilerParams(
            dimension_semantics=("arbitrary",),
        ),
    )(A, h16)

    wt = W.T
    wa = wt[:d]
    wb = wt[d:]

    be = 2000
    out = pl.pallas_call(
        _proj_body,
        grid=(n // be,),
        in_specs=[
            pl.BlockSpec((be, d), lambda i: (i, 0)),
            pl.BlockSpec((be, d), lambda i: (i, 0)),
            pl.BlockSpec((d, d), lambda i: (0, 0)),
            pl.BlockSpec((d, d), lambda i: (0, 0)),
            pl.BlockSpec((1, 1), lambda i: (0, 0)),
        ],
        out_specs=pl.BlockSpec((be, d), lambda i: (i, 0)),
        out_shape=jax.ShapeDtypeStruct((n, d), jnp.float32),
    )(h, u, wa, wb, s)
    return out


# speedup vs baseline: 1.0433x; 1.0433x over previous
import jax
import jax.numpy as jnp
from jax.experimental import pallas as pl
from jax.experimental.pallas import tpu as pltpu


def _agg_body(a_ref, h_ref, u_ref, s_ref):
    i = pl.program_id(0)

    @pl.when(i == 0)
    def _init_s():
        s_ref[...] = jnp.zeros_like(s_ref)

    a = a_ref[...]
    u_ref[...] = jnp.dot(a, h_ref[...], preferred_element_type=jnp.float32)
    s_ref[...] += jnp.sum(a)[None, None]


def _proj_body(h_ref, u_ref, wa_ref, wb_ref, s_ref, o_ref):
    inv = 1.0 / s_ref[0, 0]
    o = jnp.dot(h_ref[...], wa_ref[...], preferred_element_type=jnp.float32)
    o += jnp.dot(u_ref[...], wb_ref[...], preferred_element_type=jnp.float32) * inv
    o_ref[...] = jnp.maximum(o, 0.0)


def kernel(h, A, W):
    n, d = h.shape
    bi = 400

    u, s = pl.pallas_call(
        _agg_body,
        grid=(n // bi,),
        in_specs=[
            pl.BlockSpec((bi, n), lambda i: (i, 0)),
            pl.BlockSpec((n, d), lambda i: (0, 0)),
        ],
        out_specs=[
            pl.BlockSpec((bi, d), lambda i: (i, 0)),
            pl.BlockSpec((1, 1), lambda i: (0, 0)),
        ],
        out_shape=[
            jax.ShapeDtypeStruct((n, d), jnp.float32),
            jax.ShapeDtypeStruct((1, 1), jnp.float32),
        ],
        compiler_params=pltpu.CompilerParams(
            dimension_semantics=("arbitrary",),
        ),
    )(A, h)

    wt = W.T
    wa = wt[:d]
    wb = wt[d:]

    be = 2000
    out = pl.pallas_call(
        _proj_body,
        grid=(n // be,),
        in_specs=[
            pl.BlockSpec((be, d), lambda i: (i, 0)),
            pl.BlockSpec((be, d), lambda i: (i, 0)),
            pl.BlockSpec((d, d), lambda i: (0, 0)),
            pl.BlockSpec((d, d), lambda i: (0, 0)),
            pl.BlockSpec((1, 1), lambda i: (0, 0)),
        ],
        out_specs=pl.BlockSpec((be, d), lambda i: (i, 0)),
        out_shape=jax.ShapeDtypeStruct((n, d), jnp.float32),
    )(h, u, wa, wb, s)
    return out


# speedup vs baseline: 1.0597x; 1.0158x over previous
import jax
import jax.numpy as jnp
from jax.experimental import pallas as pl
from jax.experimental.pallas import tpu as pltpu


def _agg_body(a_ref, h_ref, u_ref, s_ref):
    i = pl.program_id(0)

    @pl.when(i == 0)
    def _init_s():
        s_ref[...] = jnp.zeros_like(s_ref)

    d = u_ref.shape[1]
    u_aug = jnp.dot(a_ref[...], h_ref[...], preferred_element_type=jnp.float32)
    u_ref[...] = u_aug[:, :d]
    s_ref[...] += jnp.sum(u_aug[:, d])[None, None]


def _proj_body(h_ref, u_ref, wa_ref, wb_ref, s_ref, o_ref):
    inv = 1.0 / s_ref[0, 0]
    o = jnp.dot(h_ref[...], wa_ref[...], preferred_element_type=jnp.float32)
    o += jnp.dot(u_ref[...], wb_ref[...], preferred_element_type=jnp.float32) * inv
    o_ref[...] = jnp.maximum(o, 0.0)


def kernel(h, A, W):
    n, d = h.shape
    bi = 400

    daug = d + 8
    h_aug = jnp.concatenate(
        [h, jnp.ones((n, 1), jnp.float32), jnp.zeros((n, 7), jnp.float32)],
        axis=1,
    )
    u, s = pl.pallas_call(
        _agg_body,
        grid=(n // bi,),
        in_specs=[
            pl.BlockSpec((bi, n), lambda i: (i, 0)),
            pl.BlockSpec((n, daug), lambda i: (0, 0)),
        ],
        out_specs=[
            pl.BlockSpec((bi, d), lambda i: (i, 0)),
            pl.BlockSpec((1, 1), lambda i: (0, 0)),
        ],
        out_shape=[
            jax.ShapeDtypeStruct((n, d), jnp.float32),
            jax.ShapeDtypeStruct((1, 1), jnp.float32),
        ],
        compiler_params=pltpu.CompilerParams(
            dimension_semantics=("arbitrary",),
        ),
    )(A, h_aug)

    wt = W.T
    wa = wt[:d]
    wb = wt[d:]

    be = 2000
    out = pl.pallas_call(
        _proj_body,
        grid=(n // be,),
        in_specs=[
            pl.BlockSpec((be, d), lambda i: (i, 0)),
            pl.BlockSpec((be, d), lambda i: (i, 0)),
            pl.BlockSpec((d, d), lambda i: (0, 0)),
            pl.BlockSpec((d, d), lambda i: (0, 0)),
            pl.BlockSpec((1, 1), lambda i: (0, 0)),
        ],
        out_specs=pl.BlockSpec((be, d), lambda i: (i, 0)),
        out_shape=jax.ShapeDtypeStruct((n, d), jnp.float32),
    )(h, u, wa, wb, s)
    return out


# speedup vs baseline: 1.0687x; 1.0084x over previous
import jax
import jax.numpy as jnp
from jax.experimental import pallas as pl
from jax.experimental.pallas import tpu as pltpu


def _fused_body(a_ref, h_ref, wa_ref, wb_ref, o_ref, p_ref, q_ref, s_ref):
    i = pl.program_id(0)
    ni = pl.num_programs(0)
    bi = a_ref.shape[0]
    d = wa_ref.shape[0]
    r0 = i * bi

    @pl.when(i == 0)
    def _init_s():
        s_ref[...] = jnp.zeros_like(s_ref)

    u_aug = jnp.dot(a_ref[...], h_ref[...], preferred_element_type=jnp.float32)
    s_ref[...] += jnp.sum(u_aug[:, d])[None, None]
    q_ref[pl.ds(r0, bi), :] = jnp.dot(
        u_aug[:, :d], wb_ref[...], preferred_element_type=jnp.float32
    )
    p_ref[pl.ds(r0, bi), :] = jnp.dot(
        h_ref[pl.ds(r0, bi), :d], wa_ref[...], preferred_element_type=jnp.float32
    )

    @pl.when(i == ni - 1)
    def _finale():
        inv = 1.0 / s_ref[0, 0]
        o_ref[...] = jnp.maximum(p_ref[...] + q_ref[...] * inv, 0.0)


def kernel(h, A, W):
    n, d = h.shape
    bi = 200
    daug = d + 8
    h_aug = jnp.concatenate(
        [h, jnp.ones((n, 1), jnp.float32), jnp.zeros((n, 7), jnp.float32)],
        axis=1,
    )
    wt = W.T
    wa = wt[:d]
    wb = wt[d:]

    out = pl.pallas_call(
        _fused_body,
        grid=(n // bi,),
        in_specs=[
            pl.BlockSpec((bi, n), lambda i: (i, 0)),
            pl.BlockSpec((n, daug), lambda i: (0, 0)),
            pl.BlockSpec((d, d), lambda i: (0, 0)),
            pl.BlockSpec((d, d), lambda i: (0, 0)),
        ],
        out_specs=pl.BlockSpec((n, d), lambda i: (0, 0)),
        out_shape=jax.ShapeDtypeStruct((n, d), jnp.float32),
        scratch_shapes=[
            pltpu.VMEM((n, d), jnp.float32),
            pltpu.VMEM((n, d), jnp.float32),
            pltpu.VMEM((1, 1), jnp.float32),
        ],
        compiler_params=pltpu.CompilerParams(
            dimension_semantics=("arbitrary",),
        ),
    )(A, h_aug, wa, wb)
    return out


# speedup vs baseline: 1.1087x; 1.0375x over previous
import jax
import jax.numpy as jnp
from jax.experimental import pallas as pl
from jax.experimental.pallas import tpu as pltpu


def _fused_body(a_ref, h_ref, wa_ref, wb_ref, o_ref, hg_ref, p_ref, q_ref, s_ref):
    i = pl.program_id(0)
    ni = pl.num_programs(0) - (p_ref.shape[0] // o_ref.shape[0])
    bi = a_ref.shape[0]
    d = wa_ref.shape[0]

    @pl.when(i == 0)
    def _init():
        s_ref[...] = jnp.zeros_like(s_ref)
        hg_ref[:, :d] = h_ref[...]
        hg_ref[:, d:] = jnp.ones_like(hg_ref[:, d:])

    @pl.when(i < ni)
    def _stripe():
        r0 = i * bi
        u_aug = jnp.dot(
            a_ref[...], hg_ref[...], preferred_element_type=jnp.float32
        )
        s_ref[...] += jnp.sum(u_aug[:, d])[None, None]
        q_ref[pl.ds(r0, bi), :] = jnp.dot(
            u_aug[:, :d], wb_ref[...], preferred_element_type=jnp.float32
        )
        p_ref[pl.ds(r0, bi), :] = jnp.dot(
            h_ref[pl.ds(r0, bi), :], wa_ref[...], preferred_element_type=jnp.float32
        )

    @pl.when(i >= ni)
    def _finale():
        ce = o_ref.shape[0]
        c0 = (i - ni) * ce
        inv = 1.0 / s_ref[0, 0]
        o_ref[...] = jnp.maximum(
            p_ref[pl.ds(c0, ce), :] + q_ref[pl.ds(c0, ce), :] * inv, 0.0
        )


def kernel(h, A, W):
    n, d = h.shape
    bi = 200
    ni = n // bi
    k = 5
    ce = n // k
    daug = d + 8

    wt = W.T
    wa = wt[:d]
    wb = wt[d:]

    out = pl.pallas_call(
        _fused_body,
        grid=(ni + k,),
        in_specs=[
            pl.BlockSpec((bi, n), lambda i: (jnp.minimum(i, ni - 1), 0)),
            pl.BlockSpec((n, d), lambda i: (0, 0)),
            pl.BlockSpec((d, d), lambda i: (0, 0)),
            pl.BlockSpec((d, d), lambda i: (0, 0)),
        ],
        out_specs=pl.BlockSpec((ce, d), lambda i: (jnp.maximum(i - ni, 0), 0)),
        out_shape=jax.ShapeDtypeStruct((n, d), jnp.float32),
        scratch_shapes=[
            pltpu.VMEM((n, daug), jnp.float32),
            pltpu.VMEM((n, d), jnp.float32),
            pltpu.VMEM((n, d), jnp.float32),
            pltpu.VMEM((1, 1), jnp.float32),
        ],
        compiler_params=pltpu.CompilerParams(
            dimension_semantics=("arbitrary",),
        ),
    )(A, h, wa, wb)
    return out


# speedup vs baseline: 1.1428x; 1.0307x over previous
import jax
import jax.numpy as jnp
from jax.experimental import pallas as pl
from jax.experimental.pallas import tpu as pltpu

_NBUF = 3


def _fused_body(
    a_hbm, h_ref, wa_ref, wb_ref, o_ref, abuf, hg_ref, p_ref, q_ref, s_ref, sems
):
    i = pl.program_id(0)
    ni = pl.num_programs(0) - (p_ref.shape[0] // o_ref.shape[0])
    bi = abuf.shape[1]
    d = wa_ref.shape[0]

    def stripe_copy(stripe, slot):
        return pltpu.make_async_copy(
            a_hbm.at[pl.ds(stripe * bi, bi), :],
            abuf.at[slot],
            sems.at[slot],
        )

    @pl.when(i == 0)
    def _init():
        s_ref[...] = jnp.zeros_like(s_ref)
        hg_ref[:, :d] = h_ref[...]
        hg_ref[:, d:] = jnp.ones_like(hg_ref[:, d:])
        for b in range(1, _NBUF):
            stripe_copy(b, b).start()

    @pl.when(i < ni)
    def _stripe():
        slot = jax.lax.rem(i, _NBUF)
        @pl.when(i == 0)
        def _first():
            stripe_copy(0, 0).start()

        stripe_copy(i, slot).wait()
        r0 = i * bi
        a = abuf[slot]
        u_aug = jnp.dot(a, hg_ref[...], preferred_element_type=jnp.float32)
        s_ref[...] += jnp.sum(u_aug[:, d])[None, None]
        q_ref[pl.ds(r0, bi), :] = jnp.dot(
            u_aug[:, :d], wb_ref[...], preferred_element_type=jnp.float32
        )
        p_ref[pl.ds(r0, bi), :] = jnp.dot(
            h_ref[pl.ds(r0, bi), :], wa_ref[...], preferred_element_type=jnp.float32
        )

        @pl.when(i + _NBUF < ni)
        def _prefetch():
            stripe_copy(i + _NBUF, slot).start()

    @pl.when(i >= ni)
    def _finale():
        ce = o_ref.shape[0]
        c0 = (i - ni) * ce
        inv = 1.0 / s_ref[0, 0]
        o_ref[...] = jnp.maximum(
            p_ref[pl.ds(c0, ce), :] + q_ref[pl.ds(c0, ce), :] * inv, 0.0
        )


def kernel(h, A, W):
    n, d = h.shape
    bi = 200
    ni = n // bi
    k = 5
    ce = n // k
    daug = d + 8

    wt = W.T
    wa = wt[:d]
    wb = wt[d:]

    out = pl.pallas_call(
        _fused_body,
        grid=(ni + k,),
        in_specs=[
            pl.BlockSpec(memory_space=pltpu.MemorySpace.HBM),
            pl.BlockSpec((n, d), lambda i: (0, 0)),
            pl.BlockSpec((d, d), lambda i: (0, 0)),
            pl.BlockSpec((d, d), lambda i: (0, 0)),
        ],
        out_specs=pl.BlockSpec((ce, d), lambda i: (jnp.maximum(i - ni, 0), 0)),
        out_shape=jax.ShapeDtypeStruct((n, d), jnp.float32),
        scratch_shapes=[
            pltpu.VMEM((_NBUF, bi, n), jnp.float32),
            pltpu.VMEM((n, daug), jnp.float32),
            pltpu.VMEM((n, d), jnp.float32),
            pltpu.VMEM((n, d), jnp.float32),
            pltpu.VMEM((1, 1), jnp.float32),
            pltpu.SemaphoreType.DMA((_NBUF,)),
        ],
        compiler_params=pltpu.CompilerParams(
            dimension_semantics=("arbitrary",),
        ),
    )(A, h, wa, wb)
    return out
